# baseline (speedup 1.0000x reference)
import functools

import jax
import jax.numpy as jnp
from jax import lax
from jax.experimental import pallas as pl
from jax.experimental.pallas import tpu as pltpu
from jax.experimental.pallas import tpu_sc as plsc

_DP = 128
_L = 16


@functools.cache
def _make_lookup(BATCH: int, SEQ: int, D: int):
    info = plsc.get_sparse_core_info()
    NC, NS = info.num_cores, info.num_subcores
    NW = NC * NS
    assert BATCH % (NW * _DP) == 0 and D % _L == 0 and SEQ % 2 == 0
    BB = BATCH // NW
    n_idx = BB * SEQ
    mesh = plsc.VectorSubcoreMesh(core_axis_name="c", subcore_axis_name="s")

    @functools.partial(
        pl.kernel,
        out_type=jax.ShapeDtypeStruct((SEQ, D, BATCH), jnp.float32),
        mesh=mesh,
        scratch_types=[
            pltpu.VMEM((n_idx,), jnp.int32),
            pltpu.VMEM((SEQ, _DP), jnp.int32),
            pltpu.VMEM((2, _DP, _DP), jnp.float32),
            pltpu.VMEM((2, D, _DP), jnp.float32),
            pltpu.SemaphoreType.DMA,
            pltpu.SemaphoreType.DMA,
            pltpu.SemaphoreType.DMA,
            pltpu.SemaphoreType.DMA,
        ],
        compiler_params=pltpu.CompilerParams(
            use_tc_tiling_on_sc=True, needs_layout_passes=False
        ),
    )
    def lookup(table_hbm, idx_hbm, out_hbm, idx_v, idx_t, rows_v, trn_v,
               gsem0, gsem1, osem0, osem1):
        gsem = (gsem0, gsem1)
        osem = (osem0, osem1)
        wid = lax.axis_index("s") * NC + lax.axis_index("c")
        b0 = wid * BB
        pltpu.sync_copy(idx_hbm.at[pl.ds(b0 * SEQ, n_idx)], idx_v)

        lanes = lax.iota(jnp.int32, _L)
        stride_s = lanes * SEQ
        row_m = [lanes + m * _L for m in range(_DP // _L)]

        @pl.loop(0, SEQ, unroll=4)
        def _regroup(s):
            for q in range(_DP // _L):
                vec = plsc.load_gather(idx_v, [stride_s + (s + q * _L * SEQ)])
                idx_t[s, pl.ds(q * _L, _L)] = vec

        def fire_gather(s, buf):
            pltpu.async_copy(
                table_hbm.at[idx_t.at[s]],
                rows_v.at[buf],
                gsem[buf],
            )

        def wait_gather(buf):
            pltpu.make_async_copy(
                table_hbm.at[pl.ds(0, _DP)], rows_v.at[buf], gsem[buf]
            ).wait()

        def transpose(buf):
            @pl.loop(0, D, unroll=8)
            def _t(d):
                col = lanes * 0 + d
                for m in range(_DP // _L):
                    vec = plsc.load_gather(rows_v.at[buf], [row_m[m], col])
                    trn_v[buf, d, pl.ds(m * _L, _L)] = vec

        def fire_out(s, buf):
            pltpu.async_copy(
                trn_v.at[buf], out_hbm.at[s, :, pl.ds(b0, _DP)], osem[buf]
            )

        def wait_out(buf):
            pltpu.make_async_copy(
                trn_v.at[buf], out_hbm.at[0, :, pl.ds(0, _DP)], osem[buf]
            ).wait()

        fire_gather(0, 0)

        @pl.loop(0, SEQ // 2)
        def _step(g):
            for p in range(2):
                s = 2 * g + p

                if p == 0:
                    fire_gather(s + 1, 1 - p)
                else:
                    @pl.when(g < SEQ // 2 - 1)
                    def _():
                        fire_gather(s + 1, 1 - p)

                wait_gather(p)

                @pl.when(g >= 1)
                def _():
                    wait_out(p)

                transpose(p)
                fire_out(s, p)

        wait_out(0)
        wait_out(1)

    return lookup


def kernel(x, table):
    BATCH, SEQ = x.shape
    V, D = table.shape
    table_p = jnp.pad(table, ((0, 0), (0, _DP - D)))
    out_t = _make_lookup(BATCH, SEQ, D)(table_p, x.reshape(-1))
    return out_t.transpose(2, 0, 1)


# speedup vs baseline: 1.7518x; 1.7518x over previous
import functools

import jax
import jax.numpy as jnp
from jax import lax
from jax.experimental import pallas as pl
from jax.experimental.pallas import tpu as pltpu
from jax.experimental.pallas import tpu_sc as plsc

_LANE = 128
_K = 5


@functools.cache
def _make_lookup(BATCH: int, SEQ: int, V: int):
    info = plsc.get_sparse_core_info()
    NC, NS = info.num_cores, info.num_subcores
    NW = NC * NS
    assert BATCH % NW == 0
    b_per_w = BATCH // NW
    n_idx = b_per_w * SEQ
    slabs = []
    off = 0
    while off < SEQ:
        n = min(128, SEQ - off)
        slabs.append((off, n))
        off += n
    mesh = plsc.VectorSubcoreMesh(core_axis_name="c", subcore_axis_name="s")

    @functools.partial(
        pl.kernel,
        out_type=jax.ShapeDtypeStruct((BATCH, SEQ, _LANE), jnp.float32),
        mesh=mesh,
        scratch_types=[
            pltpu.VMEM((n_idx,), jnp.int32),
            pltpu.VMEM((2, SEQ, _LANE), jnp.float32),
            pltpu.SemaphoreType.DMA,
            pltpu.SemaphoreType.DMA,
        ],
        compiler_params=pltpu.CompilerParams(use_tc_tiling_on_sc=True),
    )
    def lookup(table_hbm, idx_hbm, out_hbm, idx_v, rows_v, gsem, osem):
        wid = lax.axis_index("s") * NC + lax.axis_index("c")
        b0 = wid * b_per_w
        pltpu.sync_copy(idx_hbm.at[pl.ds(b0 * SEQ, n_idx)], idx_v)

        def fire_gathers(i, b):
            for off, n in slabs:
                pltpu.async_copy(
                    table_hbm.at[idx_v.at[pl.ds(i * SEQ + off, n)]],
                    rows_v.at[b].at[pl.ds(off, n)],
                    gsem,
                )

        def drain(sem):
            pltpu.make_async_copy(
                out_hbm.at[b0], rows_v.at[0], sem
            ).wait()

        def fire_out(i, b):
            pltpu.async_copy(rows_v.at[b], out_hbm.at[b0 + i], osem)

        fire_gathers(0, 0)

        @pl.loop(1, b_per_w)
        def _chunk(i):
            b = lax.rem(i, 2)
            @pl.when(i >= 2)
            def _():
                drain(osem)

            fire_gathers(i, b)
            drain(gsem)
            fire_out(i - 1, 1 - b)

        last = (b_per_w - 1) % 2
        drain(gsem)
        fire_out(b_per_w - 1, last)
        drain(osem)
        drain(osem)

    return lookup


def kernel(x, table):
    BATCH, SEQ = x.shape
    V, D = table.shape
    table_p = jnp.pad(table, ((0, 0), (0, _LANE - D)))
    out_p = _make_lookup(BATCH, SEQ, V)(table_p, x.reshape(-1))
    return out_p[:, :, :D]
